# Initial kernel scaffold; baseline (speedup 1.0000x reference)
import jax
import jax.numpy as jnp
from jax.experimental import pallas as pl


def kernel(x, edge_index, W1, b1, W2, b2, Wi1, bi1, Wi2, bi2, Wc1, bc1, Wc2, bc2):
    raise NotImplementedError("write your pallas kernel here")


# speedup vs baseline: 20.5483x; 20.5483x over previous
import functools

import jax
import jax.numpy as jnp
from jax import lax
from jax.experimental import pallas as pl
from jax.experimental.pallas import tpu as pltpu
from jax.experimental.pallas import tpu_sc as plsc

N = 10000
E = 320000
NFEAT = 128
NHID = 64
F1 = 2 * NHID
F1H = F1 // 2
F2 = NHID
PROHID = 128
NCLASS = 16

NC = 2
NS = 16
NT = NC * NS
K = 80
N_PAD = 10240
RPT = N_PAD // NS

_mesh = plsc.VectorSubcoreMesh(core_axis_name="c", subcore_axis_name="s")


EPT32 = E // NT


@functools.partial(
    pl.kernel,
    out_type=jax.ShapeDtypeStruct((NC, N_PAD), jnp.float32),
    mesh=_mesh,
    scratch_types=[
        pltpu.VMEM((N_PAD,), jnp.float32),
        pltpu.VMEM((EPT32,), jnp.int32),
        pltpu.VMEM((RPT,), jnp.float32),
        pltpu.VMEM((RPT,), jnp.float32),
        pltpu.VMEM_SHARED((NS, N_PAD), jnp.float32),
    ],
    compiler_params=pltpu.CompilerParams(needs_layout_passes=False),
)
def _sc_deg(dst_hbm, out_hbm, hist, idxv, acc, tmp, spm):
    c = lax.axis_index("c")
    s = lax.axis_index("s")
    w = c * NS + s

    zero16 = jnp.zeros((16,), jnp.float32)
    one16 = jnp.full((16,), 1.0, jnp.float32)

    @pl.loop(0, N_PAD // 16)
    def _(i):
        hist[pl.ds(i * 16, 16)] = zero16

    pltpu.sync_copy(dst_hbm.at[pl.ds(w * EPT32, EPT32)], idxv)

    @pl.loop(0, EPT32 // 16)
    def _(i):
        idx = idxv[pl.ds(i * 16, 16)]
        plsc.addupdate_scatter(hist, [idx], one16)

    pltpu.sync_copy(hist, spm.at[s])
    plsc.subcore_barrier()

    @pl.loop(0, RPT // 16)
    def _(i):
        acc[pl.ds(i * 16, 16)] = zero16

    for t in range(NS):
        pltpu.sync_copy(spm.at[t, pl.ds(s * RPT, RPT)], tmp)

        @pl.loop(0, RPT // 16)
        def _(i):
            sl = pl.ds(i * 16, 16)
            acc[sl] = acc[sl] + tmp[sl]

    pltpu.sync_copy(acc, out_hbm.at[c, pl.ds(s * RPT, RPT)])


def _make_sc_agg(Fh, col_split):
    ept = E // NS if col_split else E // NT
    nch = ept // K

    @functools.partial(
        pl.kernel,
        out_type=jax.ShapeDtypeStruct((NC, N_PAD, Fh), jnp.float32),
        mesh=_mesh,
        scratch_types=[
            pltpu.VMEM((K,), jnp.int32), pltpu.VMEM((K,), jnp.int32),
            pltpu.VMEM((K,), jnp.int32), pltpu.VMEM((K,), jnp.int32),
            pltpu.VMEM((K, Fh), jnp.float32), pltpu.VMEM((K, Fh), jnp.float32),
            pltpu.VMEM((RPT, Fh), jnp.float32),
            pltpu.VMEM_SHARED((N_PAD, Fh), jnp.float32),
            pltpu.SemaphoreType.DMA, pltpu.SemaphoreType.DMA,
            pltpu.SemaphoreType.DMA, pltpu.SemaphoreType.DMA,
            pltpu.SemaphoreType.DMA, pltpu.SemaphoreType.DMA,
            pltpu.SemaphoreType.DMA, pltpu.SemaphoreType.DMA,
        ],
        compiler_params=pltpu.CompilerParams(use_tc_tiling_on_sc=False),
    )
    def _sc_agg(y_hbm, src_hbm, dst_hbm, out_hbm,
                si0, si1, di0, di1, r0, r1, zbuf, acc,
                ssi0, ssi1, sdi0, sdi1, sg0, sg1, ss0, ss1):
        c = lax.axis_index("c")
        s = lax.axis_index("s")
        if col_split:
            ebase = s * ept
            ytab = y_hbm.at[c]
        else:
            ebase = (c * NS + s) * ept
            ytab = y_hbm

        sidx = (si0, si1)
        didx = (di0, di1)
        rows = (r0, r1)
        ssi = (ssi0, ssi1)
        sdi = (sdi0, sdi1)
        sg = (sg0, sg1)
        ss = (ss0, ss1)

        zero16 = jnp.zeros((16,), jnp.float32)

        @pl.loop(0, RPT)
        def _(i):
            for j in range(Fh // 16):
                zbuf[i, pl.ds(j * 16, 16)] = zero16

        pltpu.sync_copy(zbuf, acc.at[pl.ds(s * RPT, RPT)])
        plsc.subcore_barrier()

        def start_idx(g, b):
            off = ebase + g * K
            pltpu.async_copy(src_hbm.at[pl.ds(off, K)], sidx[b], ssi[b])
            pltpu.async_copy(dst_hbm.at[pl.ds(off, K)], didx[b], sdi[b])

        def wait_idx(b):
            pltpu.make_async_copy(src_hbm.at[pl.ds(0, K)], sidx[b], ssi[b]).wait()
            pltpu.make_async_copy(dst_hbm.at[pl.ds(0, K)], didx[b], sdi[b]).wait()

        def start_gather(b):
            pltpu.async_copy(ytab.at[sidx[b]], rows[b], sg[b])

        def wait_gather(b):
            pltpu.make_async_copy(ytab.at[sidx[b]], rows[b], sg[b]).wait()

        def start_scatter(b):
            pltpu.async_copy(rows[b], acc.at[didx[b]], ss[b], add=True)

        def wait_scatter(b):
            pltpu.make_async_copy(rows[b], acc.at[didx[b]], ss[b]).wait()

        start_idx(0, 0)
        start_idx(1, 1)
        wait_idx(0)
        start_gather(0)

        def body(g, b, last):
            nb = 1 - b
            wait_gather(b)
            start_scatter(b)
            if not last:
                wait_idx(nb)
                start_gather(nb)
            wait_scatter(b)

            @pl.when(g + 2 < nch)
            def _():
                start_idx(g + 2, b)

        if nch % 2:
            @pl.loop(0, (nch - 1) // 2)
            def _(gg):
                g = gg * 2
                body(g, 0, False)
                body(g + 1, 1, False)

            body(nch - 1, 0, True)
        else:
            @pl.loop(0, (nch - 2) // 2)
            def _(gg):
                g = gg * 2
                body(g, 0, False)
                body(g + 1, 1, False)

            body(nch - 2, 0, False)
            body(nch - 1, 1, True)

        plsc.subcore_barrier()
        pltpu.sync_copy(acc.at[pl.ds(s * RPT, RPT)],
                        out_hbm.at[c, pl.ds(s * RPT, RPT)])

    return _sc_agg


_sc_agg1 = _make_sc_agg(F1H, col_split=True)
_sc_agg2 = _make_sc_agg(F2, col_split=False)


BLK = 1000
GRID = N // BLK


def _tc1_body(deg_ref, x_ref, w_ref, y_ref, dis_ref):
    d = deg_ref[...]
    dis = lax.rsqrt(d[0] + d[1] + 1.0)
    xw = jnp.dot(x_ref[...], w_ref[...], preferred_element_type=jnp.float32)
    y = dis * xw
    y_ref[0] = y[:, :F1H]
    y_ref[1] = y[:, F1H:]
    dis_ref[...] = dis


def _tc1(deg_col, x, W1):
    return pl.pallas_call(
        _tc1_body,
        grid=(GRID,),
        in_specs=[
            pl.BlockSpec((NC, BLK, 1), lambda i: (0, i, 0)),
            pl.BlockSpec((BLK, NFEAT), lambda i: (i, 0)),
            pl.BlockSpec((NFEAT, F1), lambda i: (0, 0)),
        ],
        out_specs=[
            pl.BlockSpec((NC, BLK, F1H), lambda i: (0, i, 0)),
            pl.BlockSpec((BLK, 1), lambda i: (i, 0)),
        ],
        out_shape=[
            jax.ShapeDtypeStruct((NC, N, F1H), jnp.float32),
            jax.ShapeDtypeStruct((N, 1), jnp.float32),
        ],
    )(deg_col, x, W1)


def _tc2_body(agg_ref, y1_ref, dis_ref, b1_ref, w2_ref, y2_ref):
    a = agg_ref[...]
    y = y1_ref[...]
    dis = dis_ref[...]
    b1 = b1_ref[...]
    h1a = jax.nn.relu(dis * (a[0] + y[0]) + b1[:, :F1H])
    h1b = jax.nn.relu(dis * (a[1] + y[1]) + b1[:, F1H:])
    w2 = w2_ref[...]
    xw = (jnp.dot(h1a, w2[:F1H], preferred_element_type=jnp.float32)
          + jnp.dot(h1b, w2[F1H:], preferred_element_type=jnp.float32))
    y2_ref[...] = dis * xw


def _tc2(agg1, y1, dis, b1, W2):
    return pl.pallas_call(
        _tc2_body,
        grid=(GRID,),
        in_specs=[
            pl.BlockSpec((NC, BLK, F1H), lambda i: (0, i, 0)),
            pl.BlockSpec((NC, BLK, F1H), lambda i: (0, i, 0)),
            pl.BlockSpec((BLK, 1), lambda i: (i, 0)),
            pl.BlockSpec((1, F1), lambda i: (0, 0)),
            pl.BlockSpec((F1, F2), lambda i: (0, 0)),
        ],
        out_specs=pl.BlockSpec((BLK, F2), lambda i: (i, 0)),
        out_shape=jax.ShapeDtypeStruct((N, F2), jnp.float32),
    )(agg1, y1, dis, b1, W2)


def _tc3_body(agg_ref, y2_ref, dis_ref, b2_ref,
              wi1_ref, bi1_ref, wi2_ref, bi2_ref,
              wc1_ref, bc1_ref, wc2_ref, bc2_ref,
              z_ref, c_ref):
    a = agg_ref[...]
    dis = dis_ref[...]
    h = jax.nn.relu(dis * (a[0] + a[1] + y2_ref[...]) + b2_ref[...])

    t = jax.nn.relu(jnp.dot(h, wi1_ref[...], preferred_element_type=jnp.float32)
                    + bi1_ref[...])
    zi = jnp.dot(t, wi2_ref[...], preferred_element_type=jnp.float32) + bi2_ref[...]
    nrm = jnp.sqrt(jnp.sum(zi * zi, axis=1, keepdims=True))
    z_ref[...] = zi / jnp.maximum(nrm, 1e-12)

    u = jax.nn.relu(jnp.dot(h, wc1_ref[...], preferred_element_type=jnp.float32)
                    + bc1_ref[...])
    lg = jnp.dot(u, wc2_ref[...], preferred_element_type=jnp.float32) + bc2_ref[...]
    m = jnp.max(lg, axis=1, keepdims=True)
    ex = jnp.exp(lg - m)
    c_ref[...] = ex / jnp.sum(ex, axis=1, keepdims=True)


def _tc3(agg2, y2, dis, b2, Wi1, bi1, Wi2, bi2, Wc1, bc1, Wc2, bc2):
    return pl.pallas_call(
        _tc3_body,
        grid=(GRID,),
        in_specs=[
            pl.BlockSpec((NC, BLK, F2), lambda i: (0, i, 0)),
            pl.BlockSpec((BLK, F2), lambda i: (i, 0)),
            pl.BlockSpec((BLK, 1), lambda i: (i, 0)),
            pl.BlockSpec((1, F2), lambda i: (0, 0)),
            pl.BlockSpec((F2, PROHID), lambda i: (0, 0)),
            pl.BlockSpec((1, PROHID), lambda i: (0, 0)),
            pl.BlockSpec((PROHID, F2), lambda i: (0, 0)),
            pl.BlockSpec((1, F2), lambda i: (0, 0)),
            pl.BlockSpec((F2, PROHID), lambda i: (0, 0)),
            pl.BlockSpec((1, PROHID), lambda i: (0, 0)),
            pl.BlockSpec((PROHID, NCLASS), lambda i: (0, 0)),
            pl.BlockSpec((1, NCLASS), lambda i: (0, 0)),
        ],
        out_specs=[
            pl.BlockSpec((BLK, F2), lambda i: (i, 0)),
            pl.BlockSpec((BLK, NCLASS), lambda i: (i, 0)),
        ],
        out_shape=[
            jax.ShapeDtypeStruct((N, F2), jnp.float32),
            jax.ShapeDtypeStruct((N, NCLASS), jnp.float32),
        ],
    )(agg2, y2, dis, b2, Wi1, bi1, Wi2, bi2, Wc1, bc1, Wc2, bc2)


@jax.jit
def kernel(x, edge_index, W1, b1, W2, b2, Wi1, bi1, Wi2, bi2, Wc1, bc1, Wc2, bc2):
    src = edge_index[0]
    dst = edge_index[1]

    deg_p = _sc_deg(dst)
    deg_col = deg_p[:, :N, None]

    y1, dis = _tc1(deg_col, x, W1)
    agg1 = _sc_agg1(y1, src, dst)[:, :N]
    y2 = _tc2(agg1, y1, dis, b1.reshape(1, F1), W2)
    agg2 = _sc_agg2(y2, src, dst)[:, :N]
    z, c = _tc3(agg2, y2, dis, b2.reshape(1, F2),
                Wi1, bi1.reshape(1, PROHID), Wi2, bi2.reshape(1, F2),
                Wc1, bc1.reshape(1, PROHID), Wc2, bc2.reshape(1, NCLASS))
    return (z, c)


# speedup vs baseline: 20.6487x; 1.0049x over previous
import functools

import jax
import jax.numpy as jnp
from jax import lax
from jax.experimental import pallas as pl
from jax.experimental.pallas import tpu as pltpu
from jax.experimental.pallas import tpu_sc as plsc

N = 10000
E = 320000
NFEAT = 128
NHID = 64
F1 = 2 * NHID
F1H = F1 // 2
F2 = NHID
PROHID = 128
NCLASS = 16

NC = 2
NS = 16
NT = NC * NS
K = 80
N_PAD = 10240
RPT = N_PAD // NS

_mesh = plsc.VectorSubcoreMesh(core_axis_name="c", subcore_axis_name="s")


EPT32 = E // NT


@functools.partial(
    pl.kernel,
    out_type=jax.ShapeDtypeStruct((NC, N_PAD), jnp.float32),
    mesh=_mesh,
    scratch_types=[
        pltpu.VMEM((N_PAD,), jnp.float32),
        pltpu.VMEM((EPT32,), jnp.int32),
        pltpu.VMEM((RPT,), jnp.float32),
        pltpu.VMEM((RPT,), jnp.float32),
        pltpu.VMEM_SHARED((NS, N_PAD), jnp.float32),
    ],
    compiler_params=pltpu.CompilerParams(needs_layout_passes=False),
)
def _sc_deg(dst_hbm, out_hbm, hist, idxv, acc, tmp, spm):
    c = lax.axis_index("c")
    s = lax.axis_index("s")
    w = c * NS + s

    zero16 = jnp.zeros((16,), jnp.float32)
    one16 = jnp.full((16,), 1.0, jnp.float32)

    @pl.loop(0, N_PAD // 16)
    def _(i):
        hist[pl.ds(i * 16, 16)] = zero16

    pltpu.sync_copy(dst_hbm.at[pl.ds(w * EPT32, EPT32)], idxv)

    @pl.loop(0, EPT32 // 16)
    def _(i):
        idx = idxv[pl.ds(i * 16, 16)]
        plsc.addupdate_scatter(hist, [idx], one16)

    pltpu.sync_copy(hist, spm.at[s])
    plsc.subcore_barrier()

    @pl.loop(0, RPT // 16)
    def _(i):
        acc[pl.ds(i * 16, 16)] = zero16

    for t in range(NS):
        pltpu.sync_copy(spm.at[t, pl.ds(s * RPT, RPT)], tmp)

        @pl.loop(0, RPT // 16)
        def _(i):
            sl = pl.ds(i * 16, 16)
            acc[sl] = acc[sl] + tmp[sl]

    pltpu.sync_copy(acc, out_hbm.at[c, pl.ds(s * RPT, RPT)])


def _make_sc_agg(Fh, col_split):
    ept = E // NS if col_split else E // NT
    nch = ept // K
    ZR = 160

    @functools.partial(
        pl.kernel,
        out_type=jax.ShapeDtypeStruct((NC, N_PAD, Fh), jnp.float32),
        mesh=_mesh,
        scratch_types=[
            pltpu.VMEM((nch, K), jnp.int32),
            pltpu.VMEM((nch, K), jnp.int32),
            pltpu.VMEM((K, Fh), jnp.float32), pltpu.VMEM((K, Fh), jnp.float32),
            pltpu.VMEM((ZR, Fh), jnp.float32),
            pltpu.VMEM_SHARED((N_PAD, Fh), jnp.float32),
            pltpu.SemaphoreType.DMA, pltpu.SemaphoreType.DMA,
            pltpu.SemaphoreType.DMA, pltpu.SemaphoreType.DMA,
        ],
        compiler_params=pltpu.CompilerParams(use_tc_tiling_on_sc=False),
    )
    def _sc_agg(y_hbm, src2d_hbm, dst2d_hbm, out_hbm,
                sidx, didx, r0, r1, zbuf, acc,
                sg0, sg1, ss0, ss1):
        c = lax.axis_index("c")
        s = lax.axis_index("s")
        if col_split:
            rowbase = s * nch
            ytab = y_hbm.at[c]
        else:
            rowbase = (c * NS + s) * nch
            ytab = y_hbm

        rows = (r0, r1)
        sg = (sg0, sg1)
        ss = (ss0, ss1)

        pltpu.sync_copy(src2d_hbm.at[pl.ds(rowbase, nch)], sidx)
        pltpu.sync_copy(dst2d_hbm.at[pl.ds(rowbase, nch)], didx)

        zero16 = jnp.zeros((16,), jnp.float32)

        @pl.loop(0, ZR)
        def _(i):
            for j in range(Fh // 16):
                zbuf[i, pl.ds(j * 16, 16)] = zero16

        for j in range(RPT // ZR):
            pltpu.sync_copy(zbuf, acc.at[pl.ds(s * RPT + j * ZR, ZR)])
        plsc.subcore_barrier()

        def start_gather(g, b):
            pltpu.async_copy(ytab.at[sidx.at[g]], rows[b], sg[b])

        def wait_gather(g, b):
            pltpu.make_async_copy(ytab.at[sidx.at[g]], rows[b], sg[b]).wait()

        def start_scatter(g, b):
            pltpu.async_copy(rows[b], acc.at[didx.at[g]], ss[b], add=True)

        def wait_scatter(g, b):
            pltpu.make_async_copy(rows[b], acc.at[didx.at[g]], ss[b]).wait()

        start_gather(0, 0)

        def body(g, b, last):
            wait_gather(g, b)
            start_scatter(g, b)
            if not last:
                start_gather(g + 1, 1 - b)
            wait_scatter(g, b)

        if nch % 2:
            @pl.loop(0, (nch - 1) // 2)
            def _(gg):
                g = gg * 2
                body(g, 0, False)
                body(g + 1, 1, False)

            body(nch - 1, 0, True)
        else:
            @pl.loop(0, (nch - 2) // 2)
            def _(gg):
                g = gg * 2
                body(g, 0, False)
                body(g + 1, 1, False)

            body(nch - 2, 0, False)
            body(nch - 1, 1, True)

        plsc.subcore_barrier()
        pltpu.sync_copy(acc.at[pl.ds(s * RPT, RPT)],
                        out_hbm.at[c, pl.ds(s * RPT, RPT)])

    return _sc_agg


_sc_agg1 = _make_sc_agg(F1H, col_split=True)
_sc_agg2 = _make_sc_agg(F2, col_split=False)


BLK = 1000
GRID = N // BLK


def _tc1_body(deg_ref, x_ref, w_ref, y_ref, dis_ref):
    d = deg_ref[...]
    dis = lax.rsqrt(d[0] + d[1] + 1.0)
    xw = jnp.dot(x_ref[...], w_ref[...], preferred_element_type=jnp.float32)
    y = dis * xw
    y_ref[0] = y[:, :F1H]
    y_ref[1] = y[:, F1H:]
    dis_ref[...] = dis


def _tc1(deg_col, x, W1):
    return pl.pallas_call(
        _tc1_body,
        grid=(GRID,),
        in_specs=[
            pl.BlockSpec((NC, BLK, 1), lambda i: (0, i, 0)),
            pl.BlockSpec((BLK, NFEAT), lambda i: (i, 0)),
            pl.BlockSpec((NFEAT, F1), lambda i: (0, 0)),
        ],
        out_specs=[
            pl.BlockSpec((NC, BLK, F1H), lambda i: (0, i, 0)),
            pl.BlockSpec((BLK, 1), lambda i: (i, 0)),
        ],
        out_shape=[
            jax.ShapeDtypeStruct((NC, N, F1H), jnp.float32),
            jax.ShapeDtypeStruct((N, 1), jnp.float32),
        ],
    )(deg_col, x, W1)


def _tc2_body(agg_ref, y1_ref, dis_ref, b1_ref, w2_ref, y2_ref):
    a = agg_ref[...]
    y = y1_ref[...]
    dis = dis_ref[...]
    b1 = b1_ref[...]
    h1a = jax.nn.relu(dis * (a[0] + y[0]) + b1[:, :F1H])
    h1b = jax.nn.relu(dis * (a[1] + y[1]) + b1[:, F1H:])
    w2 = w2_ref[...]
    xw = (jnp.dot(h1a, w2[:F1H], preferred_element_type=jnp.float32)
          + jnp.dot(h1b, w2[F1H:], preferred_element_type=jnp.float32))
    y2_ref[...] = dis * xw


def _tc2(agg1, y1, dis, b1, W2):
    return pl.pallas_call(
        _tc2_body,
        grid=(GRID,),
        in_specs=[
            pl.BlockSpec((NC, BLK, F1H), lambda i: (0, i, 0)),
            pl.BlockSpec((NC, BLK, F1H), lambda i: (0, i, 0)),
            pl.BlockSpec((BLK, 1), lambda i: (i, 0)),
            pl.BlockSpec((1, F1), lambda i: (0, 0)),
            pl.BlockSpec((F1, F2), lambda i: (0, 0)),
        ],
        out_specs=pl.BlockSpec((BLK, F2), lambda i: (i, 0)),
        out_shape=jax.ShapeDtypeStruct((N, F2), jnp.float32),
    )(agg1, y1, dis, b1, W2)


def _tc3_body(agg_ref, y2_ref, dis_ref, b2_ref,
              wi1_ref, bi1_ref, wi2_ref, bi2_ref,
              wc1_ref, bc1_ref, wc2_ref, bc2_ref,
              z_ref, c_ref):
    a = agg_ref[...]
    dis = dis_ref[...]
    h = jax.nn.relu(dis * (a[0] + a[1] + y2_ref[...]) + b2_ref[...])

    t = jax.nn.relu(jnp.dot(h, wi1_ref[...], preferred_element_type=jnp.float32)
                    + bi1_ref[...])
    zi = jnp.dot(t, wi2_ref[...], preferred_element_type=jnp.float32) + bi2_ref[...]
    nrm = jnp.sqrt(jnp.sum(zi * zi, axis=1, keepdims=True))
    z_ref[...] = zi / jnp.maximum(nrm, 1e-12)

    u = jax.nn.relu(jnp.dot(h, wc1_ref[...], preferred_element_type=jnp.float32)
                    + bc1_ref[...])
    lg = jnp.dot(u, wc2_ref[...], preferred_element_type=jnp.float32) + bc2_ref[...]
    m = jnp.max(lg, axis=1, keepdims=True)
    ex = jnp.exp(lg - m)
    c_ref[...] = ex / jnp.sum(ex, axis=1, keepdims=True)


def _tc3(agg2, y2, dis, b2, Wi1, bi1, Wi2, bi2, Wc1, bc1, Wc2, bc2):
    return pl.pallas_call(
        _tc3_body,
        grid=(GRID,),
        in_specs=[
            pl.BlockSpec((NC, BLK, F2), lambda i: (0, i, 0)),
            pl.BlockSpec((BLK, F2), lambda i: (i, 0)),
            pl.BlockSpec((BLK, 1), lambda i: (i, 0)),
            pl.BlockSpec((1, F2), lambda i: (0, 0)),
            pl.BlockSpec((F2, PROHID), lambda i: (0, 0)),
            pl.BlockSpec((1, PROHID), lambda i: (0, 0)),
            pl.BlockSpec((PROHID, F2), lambda i: (0, 0)),
            pl.BlockSpec((1, F2), lambda i: (0, 0)),
            pl.BlockSpec((F2, PROHID), lambda i: (0, 0)),
            pl.BlockSpec((1, PROHID), lambda i: (0, 0)),
            pl.BlockSpec((PROHID, NCLASS), lambda i: (0, 0)),
            pl.BlockSpec((1, NCLASS), lambda i: (0, 0)),
        ],
        out_specs=[
            pl.BlockSpec((BLK, F2), lambda i: (i, 0)),
            pl.BlockSpec((BLK, NCLASS), lambda i: (i, 0)),
        ],
        out_shape=[
            jax.ShapeDtypeStruct((N, F2), jnp.float32),
            jax.ShapeDtypeStruct((N, NCLASS), jnp.float32),
        ],
    )(agg2, y2, dis, b2, Wi1, bi1, Wi2, bi2, Wc1, bc1, Wc2, bc2)


@jax.jit
def kernel(x, edge_index, W1, b1, W2, b2, Wi1, bi1, Wi2, bi2, Wc1, bc1, Wc2, bc2):
    src = edge_index[0]
    dst = edge_index[1]
    src2d = src.reshape(E // K, K)
    dst2d = dst.reshape(E // K, K)

    deg_p = _sc_deg(dst)
    deg_col = deg_p[:, :N, None]

    y1, dis = _tc1(deg_col, x, W1)
    agg1 = _sc_agg1(y1, src2d, dst2d)[:, :N]
    y2 = _tc2(agg1, y1, dis, b1.reshape(1, F1), W2)
    agg2 = _sc_agg2(y2, src2d, dst2d)[:, :N]
    z, c = _tc3(agg2, y2, dis, b2.reshape(1, F2),
                Wi1, bi1.reshape(1, PROHID), Wi2, bi2.reshape(1, F2),
                Wc1, bc1.reshape(1, PROHID), Wc2, bc2.reshape(1, NCLASS))
    return (z, c)


# speedup vs baseline: 28.0927x; 1.3605x over previous
import functools

import jax
import jax.numpy as jnp
from jax import lax
from jax.experimental import pallas as pl
from jax.experimental.pallas import tpu as pltpu
from jax.experimental.pallas import tpu_sc as plsc

N = 10000
E = 320000
NFEAT = 128
NHID = 64
F1 = 2 * NHID
F1H = F1 // 2
F2 = NHID
PROHID = 128
NCLASS = 16

NC = 2
NS = 16
NT = NC * NS
K = 80
N_PAD = 10240
RPT = N_PAD // NS

_mesh = plsc.VectorSubcoreMesh(core_axis_name="c", subcore_axis_name="s")


EPT32 = E // NT


@functools.partial(
    pl.kernel,
    out_type=jax.ShapeDtypeStruct((NC, N_PAD), jnp.float32),
    mesh=_mesh,
    scratch_types=[
        pltpu.VMEM((N_PAD,), jnp.float32),
        pltpu.VMEM((EPT32,), jnp.int32),
        pltpu.VMEM((RPT,), jnp.float32),
        pltpu.VMEM((RPT,), jnp.float32),
        pltpu.VMEM_SHARED((NS, N_PAD), jnp.float32),
    ],
    compiler_params=pltpu.CompilerParams(needs_layout_passes=False),
)
def _sc_deg(dst_hbm, out_hbm, hist, idxv, acc, tmp, spm):
    c = lax.axis_index("c")
    s = lax.axis_index("s")
    w = c * NS + s

    zero16 = jnp.zeros((16,), jnp.float32)
    one16 = jnp.full((16,), 1.0, jnp.float32)

    @pl.loop(0, N_PAD // 16)
    def _(i):
        hist[pl.ds(i * 16, 16)] = zero16

    pltpu.sync_copy(dst_hbm.at[pl.ds(w * EPT32, EPT32)], idxv)

    @pl.loop(0, EPT32 // 16)
    def _(i):
        idx = idxv[pl.ds(i * 16, 16)]
        plsc.addupdate_scatter(hist, [idx], one16)

    pltpu.sync_copy(hist, spm.at[s])
    plsc.subcore_barrier()

    @pl.loop(0, RPT // 16)
    def _(i):
        acc[pl.ds(i * 16, 16)] = zero16

    for t in range(NS):
        pltpu.sync_copy(spm.at[t, pl.ds(s * RPT, RPT)], tmp)

        @pl.loop(0, RPT // 16)
        def _(i):
            sl = pl.ds(i * 16, 16)
            acc[sl] = acc[sl] + tmp[sl]

    pltpu.sync_copy(acc, out_hbm.at[c, pl.ds(s * RPT, RPT)])


def _make_sc_agg(Fh, col_split):
    ept = E // NS if col_split else E // NT
    nch = ept // K
    ZR = 160

    @functools.partial(
        pl.kernel,
        out_type=jax.ShapeDtypeStruct((NC, N_PAD, Fh), jnp.float32),
        mesh=_mesh,
        scratch_types=[
            pltpu.VMEM((nch, K), jnp.int32),
            pltpu.VMEM((nch, K), jnp.int32),
            pltpu.VMEM((K, Fh), jnp.float32), pltpu.VMEM((K, Fh), jnp.float32),
            pltpu.VMEM((K, Fh), jnp.float32), pltpu.VMEM((K, Fh), jnp.float32),
            pltpu.VMEM((ZR, Fh), jnp.float32),
            pltpu.VMEM_SHARED((N_PAD, Fh), jnp.float32),
            pltpu.SemaphoreType.DMA, pltpu.SemaphoreType.DMA,
            pltpu.SemaphoreType.DMA, pltpu.SemaphoreType.DMA,
            pltpu.SemaphoreType.DMA, pltpu.SemaphoreType.DMA,
            pltpu.SemaphoreType.DMA, pltpu.SemaphoreType.DMA,
        ],
        compiler_params=pltpu.CompilerParams(use_tc_tiling_on_sc=False),
    )
    def _sc_agg(y_hbm, src2d_hbm, dst2d_hbm, out_hbm,
                sidx, didx, r0, r1, r2, r3, zbuf, acc,
                sg0, sg1, sg2, sg3, ss0, ss1, ss2, ss3):
        c = lax.axis_index("c")
        s = lax.axis_index("s")
        if col_split:
            rowbase = s * nch
            ytab = y_hbm.at[c]
        else:
            rowbase = (c * NS + s) * nch
            ytab = y_hbm

        rows = (r0, r1, r2, r3)
        sg = (sg0, sg1, sg2, sg3)
        ss = (ss0, ss1, ss2, ss3)

        pltpu.sync_copy(src2d_hbm.at[pl.ds(rowbase, nch)], sidx)
        pltpu.sync_copy(dst2d_hbm.at[pl.ds(rowbase, nch)], didx)

        zero16 = jnp.zeros((16,), jnp.float32)

        @pl.loop(0, ZR)
        def _(i):
            for j in range(Fh // 16):
                zbuf[i, pl.ds(j * 16, 16)] = zero16

        for j in range(RPT // ZR):
            pltpu.sync_copy(zbuf, acc.at[pl.ds(s * RPT + j * ZR, ZR)])
        plsc.subcore_barrier()

        def start_gather(g, b):
            pltpu.async_copy(ytab.at[sidx.at[g]], rows[b], sg[b])

        def wait_gather(g, b):
            pltpu.make_async_copy(ytab.at[sidx.at[g]], rows[b], sg[b]).wait()

        def start_scatter(g, b):
            pltpu.async_copy(rows[b], acc.at[didx.at[g]], ss[b], add=True)

        def wait_scatter(g, b):
            pltpu.make_async_copy(rows[b], acc.at[didx.at[g]], ss[b]).wait()

        start_gather(0, 0)
        start_gather(1, 1)

        def body(g, b, wait_prev, prefetch):
            wait_gather(g, b)
            start_scatter(g, b)
            if prefetch:
                nb = (b + 2) % 4
                if wait_prev:
                    wait_scatter(g - 2, nb)
                start_gather(g + 2, nb)

        body(0, 0, False, True)
        body(1, 1, False, True)

        ngroups = (nch - 4) // 4
        tail_lo = 2 + 4 * ngroups

        @pl.loop(0, ngroups)
        def _(gg):
            g0 = gg * 4 + 2
            for j in range(4):
                body(g0 + j, (2 + j) % 4, True, True)

        for g in range(tail_lo, nch):
            body(g, g % 4, True, g + 2 < nch)

        for g in range(nch - 4, nch):
            wait_scatter(g, g % 4)

        plsc.subcore_barrier()
        pltpu.sync_copy(acc.at[pl.ds(s * RPT, RPT)],
                        out_hbm.at[c, pl.ds(s * RPT, RPT)])

    return _sc_agg


_sc_agg1 = _make_sc_agg(F1H, col_split=True)
_sc_agg2 = _make_sc_agg(F2, col_split=False)


BLK = 1000
GRID = N // BLK


def _tc1_body(deg_ref, x_ref, w_ref, y_ref, dis_ref):
    d = deg_ref[...]
    dis = lax.rsqrt(d[0] + d[1] + 1.0)
    xw = jnp.dot(x_ref[...], w_ref[...], preferred_element_type=jnp.float32)
    y = dis * xw
    y_ref[0] = y[:, :F1H]
    y_ref[1] = y[:, F1H:]
    dis_ref[...] = dis


def _tc1(deg_col, x, W1):
    return pl.pallas_call(
        _tc1_body,
        grid=(GRID,),
        in_specs=[
            pl.BlockSpec((NC, BLK, 1), lambda i: (0, i, 0)),
            pl.BlockSpec((BLK, NFEAT), lambda i: (i, 0)),
            pl.BlockSpec((NFEAT, F1), lambda i: (0, 0)),
        ],
        out_specs=[
            pl.BlockSpec((NC, BLK, F1H), lambda i: (0, i, 0)),
            pl.BlockSpec((BLK, 1), lambda i: (i, 0)),
        ],
        out_shape=[
            jax.ShapeDtypeStruct((NC, N, F1H), jnp.float32),
            jax.ShapeDtypeStruct((N, 1), jnp.float32),
        ],
    )(deg_col, x, W1)


def _tc2_body(agg_ref, y1_ref, dis_ref, b1_ref, w2_ref, y2_ref):
    a = agg_ref[...]
    y = y1_ref[...]
    dis = dis_ref[...]
    b1 = b1_ref[...]
    h1a = jax.nn.relu(dis * (a[0] + y[0]) + b1[:, :F1H])
    h1b = jax.nn.relu(dis * (a[1] + y[1]) + b1[:, F1H:])
    w2 = w2_ref[...]
    xw = (jnp.dot(h1a, w2[:F1H], preferred_element_type=jnp.float32)
          + jnp.dot(h1b, w2[F1H:], preferred_element_type=jnp.float32))
    y2_ref[...] = dis * xw


def _tc2(agg1, y1, dis, b1, W2):
    return pl.pallas_call(
        _tc2_body,
        grid=(GRID,),
        in_specs=[
            pl.BlockSpec((NC, BLK, F1H), lambda i: (0, i, 0)),
            pl.BlockSpec((NC, BLK, F1H), lambda i: (0, i, 0)),
            pl.BlockSpec((BLK, 1), lambda i: (i, 0)),
            pl.BlockSpec((1, F1), lambda i: (0, 0)),
            pl.BlockSpec((F1, F2), lambda i: (0, 0)),
        ],
        out_specs=pl.BlockSpec((BLK, F2), lambda i: (i, 0)),
        out_shape=jax.ShapeDtypeStruct((N, F2), jnp.float32),
    )(agg1, y1, dis, b1, W2)


def _tc3_body(agg_ref, y2_ref, dis_ref, b2_ref,
              wi1_ref, bi1_ref, wi2_ref, bi2_ref,
              wc1_ref, bc1_ref, wc2_ref, bc2_ref,
              z_ref, c_ref):
    a = agg_ref[...]
    dis = dis_ref[...]
    h = jax.nn.relu(dis * (a[0] + a[1] + y2_ref[...]) + b2_ref[...])

    t = jax.nn.relu(jnp.dot(h, wi1_ref[...], preferred_element_type=jnp.float32)
                    + bi1_ref[...])
    zi = jnp.dot(t, wi2_ref[...], preferred_element_type=jnp.float32) + bi2_ref[...]
    nrm = jnp.sqrt(jnp.sum(zi * zi, axis=1, keepdims=True))
    z_ref[...] = zi / jnp.maximum(nrm, 1e-12)

    u = jax.nn.relu(jnp.dot(h, wc1_ref[...], preferred_element_type=jnp.float32)
                    + bc1_ref[...])
    lg = jnp.dot(u, wc2_ref[...], preferred_element_type=jnp.float32) + bc2_ref[...]
    m = jnp.max(lg, axis=1, keepdims=True)
    ex = jnp.exp(lg - m)
    c_ref[...] = ex / jnp.sum(ex, axis=1, keepdims=True)


def _tc3(agg2, y2, dis, b2, Wi1, bi1, Wi2, bi2, Wc1, bc1, Wc2, bc2):
    return pl.pallas_call(
        _tc3_body,
        grid=(GRID,),
        in_specs=[
            pl.BlockSpec((NC, BLK, F2), lambda i: (0, i, 0)),
            pl.BlockSpec((BLK, F2), lambda i: (i, 0)),
            pl.BlockSpec((BLK, 1), lambda i: (i, 0)),
            pl.BlockSpec((1, F2), lambda i: (0, 0)),
            pl.BlockSpec((F2, PROHID), lambda i: (0, 0)),
            pl.BlockSpec((1, PROHID), lambda i: (0, 0)),
            pl.BlockSpec((PROHID, F2), lambda i: (0, 0)),
            pl.BlockSpec((1, F2), lambda i: (0, 0)),
            pl.BlockSpec((F2, PROHID), lambda i: (0, 0)),
            pl.BlockSpec((1, PROHID), lambda i: (0, 0)),
            pl.BlockSpec((PROHID, NCLASS), lambda i: (0, 0)),
            pl.BlockSpec((1, NCLASS), lambda i: (0, 0)),
        ],
        out_specs=[
            pl.BlockSpec((BLK, F2), lambda i: (i, 0)),
            pl.BlockSpec((BLK, NCLASS), lambda i: (i, 0)),
        ],
        out_shape=[
            jax.ShapeDtypeStruct((N, F2), jnp.float32),
            jax.ShapeDtypeStruct((N, NCLASS), jnp.float32),
        ],
    )(agg2, y2, dis, b2, Wi1, bi1, Wi2, bi2, Wc1, bc1, Wc2, bc2)


@jax.jit
def kernel(x, edge_index, W1, b1, W2, b2, Wi1, bi1, Wi2, bi2, Wc1, bc1, Wc2, bc2):
    src = edge_index[0]
    dst = edge_index[1]
    src2d = src.reshape(E // K, K)
    dst2d = dst.reshape(E // K, K)

    deg_p = _sc_deg(dst)
    deg_col = deg_p[:, :N, None]

    y1, dis = _tc1(deg_col, x, W1)
    agg1 = _sc_agg1(y1, src2d, dst2d)[:, :N]
    y2 = _tc2(agg1, y1, dis, b1.reshape(1, F1), W2)
    agg2 = _sc_agg2(y2, src2d, dst2d)[:, :N]
    z, c = _tc3(agg2, y2, dis, b2.reshape(1, F2),
                Wi1, bi1.reshape(1, PROHID), Wi2, bi2.reshape(1, F2),
                Wc1, bc1.reshape(1, PROHID), Wc2, bc2.reshape(1, NCLASS))
    return (z, c)


# speedup vs baseline: 30.7168x; 1.0934x over previous
import functools

import jax
import jax.numpy as jnp
from jax import lax
from jax.experimental import pallas as pl
from jax.experimental.pallas import tpu as pltpu
from jax.experimental.pallas import tpu_sc as plsc

N = 10000
E = 320000
NFEAT = 128
NHID = 64
F1 = 2 * NHID
F1H = F1 // 2
F2 = NHID
PROHID = 128
NCLASS = 16

NC = 2
NS = 16
NB = 6
PD = 3
NT = NC * NS
K = 80
N_PAD = 10240
RPT = N_PAD // NS

_mesh = plsc.VectorSubcoreMesh(core_axis_name="c", subcore_axis_name="s")


EPT32 = E // NT


@functools.partial(
    pl.kernel,
    out_type=jax.ShapeDtypeStruct((NC, N_PAD), jnp.float32),
    mesh=_mesh,
    scratch_types=[
        pltpu.VMEM((N_PAD,), jnp.float32),
        pltpu.VMEM((EPT32,), jnp.int32),
        pltpu.VMEM((RPT,), jnp.float32),
        pltpu.VMEM((RPT,), jnp.float32),
        pltpu.VMEM_SHARED((NS, N_PAD), jnp.float32),
    ],
    compiler_params=pltpu.CompilerParams(needs_layout_passes=False),
)
def _sc_deg(dst_hbm, out_hbm, hist, idxv, acc, tmp, spm):
    c = lax.axis_index("c")
    s = lax.axis_index("s")
    w = c * NS + s

    zero16 = jnp.zeros((16,), jnp.float32)
    one16 = jnp.full((16,), 1.0, jnp.float32)

    @pl.loop(0, N_PAD // 16)
    def _(i):
        hist[pl.ds(i * 16, 16)] = zero16

    pltpu.sync_copy(dst_hbm.at[pl.ds(w * EPT32, EPT32)], idxv)

    @pl.loop(0, EPT32 // 16)
    def _(i):
        idx = idxv[pl.ds(i * 16, 16)]
        plsc.addupdate_scatter(hist, [idx], one16)

    pltpu.sync_copy(hist, spm.at[s])
    plsc.subcore_barrier()

    @pl.loop(0, RPT // 16)
    def _(i):
        acc[pl.ds(i * 16, 16)] = zero16

    for t in range(NS):
        pltpu.sync_copy(spm.at[t, pl.ds(s * RPT, RPT)], tmp)

        @pl.loop(0, RPT // 16)
        def _(i):
            sl = pl.ds(i * 16, 16)
            acc[sl] = acc[sl] + tmp[sl]

    pltpu.sync_copy(acc, out_hbm.at[c, pl.ds(s * RPT, RPT)])


def _make_sc_agg(Fh, col_split):
    ept = E // NS if col_split else E // NT
    nch = ept // K
    ZR = 160

    @functools.partial(
        pl.kernel,
        out_type=jax.ShapeDtypeStruct((NC, N_PAD, Fh), jnp.float32),
        mesh=_mesh,
        scratch_types=[
            pltpu.VMEM((nch, K), jnp.int32),
            pltpu.VMEM((nch, K), jnp.int32),
            *[pltpu.VMEM((K, Fh), jnp.float32) for _ in range(NB)],
            pltpu.VMEM((ZR, Fh), jnp.float32),
            pltpu.VMEM_SHARED((N_PAD, Fh), jnp.float32),
            *[pltpu.SemaphoreType.DMA for _ in range(2 * NB)],
        ],
        compiler_params=pltpu.CompilerParams(use_tc_tiling_on_sc=False),
    )
    def _sc_agg(y_hbm, src2d_hbm, dst2d_hbm, out_hbm, sidx, didx, *rest):
        c = lax.axis_index("c")
        s = lax.axis_index("s")
        if col_split:
            rowbase = s * nch
            ytab = y_hbm.at[c]
        else:
            rowbase = (c * NS + s) * nch
            ytab = y_hbm

        rows = rest[:NB]
        zbuf = rest[NB]
        acc = rest[NB + 1]
        sg = rest[NB + 2:NB + 2 + NB]
        ss = rest[NB + 2 + NB:]

        pltpu.sync_copy(src2d_hbm.at[pl.ds(rowbase, nch)], sidx)
        pltpu.sync_copy(dst2d_hbm.at[pl.ds(rowbase, nch)], didx)

        zero16 = jnp.zeros((16,), jnp.float32)

        @pl.loop(0, ZR)
        def _(i):
            for j in range(Fh // 16):
                zbuf[i, pl.ds(j * 16, 16)] = zero16

        for j in range(RPT // ZR):
            pltpu.sync_copy(zbuf, acc.at[pl.ds(s * RPT + j * ZR, ZR)])
        plsc.subcore_barrier()

        def start_gather(g, b):
            pltpu.async_copy(ytab.at[sidx.at[g]], rows[b], sg[b])

        def wait_gather(g, b):
            pltpu.make_async_copy(ytab.at[sidx.at[g]], rows[b], sg[b]).wait()

        def start_scatter(g, b):
            pltpu.async_copy(rows[b], acc.at[didx.at[g]], ss[b], add=True)

        def wait_scatter(g, b):
            pltpu.make_async_copy(rows[b], acc.at[didx.at[g]], ss[b]).wait()

        for g in range(PD):
            start_gather(g, g)

        def body(g, b, wait_prev, prefetch):
            wait_gather(g, b)
            start_scatter(g, b)
            if prefetch:
                nb = (b + PD) % NB
                if wait_prev:
                    wait_scatter(g - (NB - PD), nb)
                start_gather(g + PD, nb)

        head = NB - PD
        for g in range(head):
            body(g, g % NB, False, True)

        ngroups = (nch - head - (NB - PD)) // NB
        tail_lo = head + NB * ngroups

        @pl.loop(0, ngroups)
        def _(gg):
            g0 = gg * NB + head
            for j in range(NB):
                body(g0 + j, (head + j) % NB, True, True)

        for g in range(tail_lo, nch):
            body(g, g % NB, True, g + PD < nch)

        for g in range(max(0, nch - NB), nch):
            wait_scatter(g, g % NB)

        plsc.subcore_barrier()
        pltpu.sync_copy(acc.at[pl.ds(s * RPT, RPT)],
                        out_hbm.at[c, pl.ds(s * RPT, RPT)])

    return _sc_agg


_sc_agg1 = _make_sc_agg(F1H, col_split=True)
_sc_agg2 = _make_sc_agg(F2, col_split=False)


BLK = 1000
GRID = N // BLK


def _tc1_body(deg_ref, x_ref, w_ref, y_ref, dis_ref):
    d = deg_ref[...]
    dis = lax.rsqrt(d[0] + d[1] + 1.0)
    xw = jnp.dot(x_ref[...], w_ref[...], preferred_element_type=jnp.float32)
    y = dis * xw
    y_ref[0] = y[:, :F1H]
    y_ref[1] = y[:, F1H:]
    dis_ref[...] = dis


def _tc1(deg_col, x, W1):
    return pl.pallas_call(
        _tc1_body,
        grid=(GRID,),
        in_specs=[
            pl.BlockSpec((NC, BLK, 1), lambda i: (0, i, 0)),
            pl.BlockSpec((BLK, NFEAT), lambda i: (i, 0)),
            pl.BlockSpec((NFEAT, F1), lambda i: (0, 0)),
        ],
        out_specs=[
            pl.BlockSpec((NC, BLK, F1H), lambda i: (0, i, 0)),
            pl.BlockSpec((BLK, 1), lambda i: (i, 0)),
        ],
        out_shape=[
            jax.ShapeDtypeStruct((NC, N, F1H), jnp.float32),
            jax.ShapeDtypeStruct((N, 1), jnp.float32),
        ],
    )(deg_col, x, W1)


def _tc2_body(agg_ref, y1_ref, dis_ref, b1_ref, w2_ref, y2_ref):
    a = agg_ref[...]
    y = y1_ref[...]
    dis = dis_ref[...]
    b1 = b1_ref[...]
    h1a = jax.nn.relu(dis * (a[0] + y[0]) + b1[:, :F1H])
    h1b = jax.nn.relu(dis * (a[1] + y[1]) + b1[:, F1H:])
    w2 = w2_ref[...]
    xw = (jnp.dot(h1a, w2[:F1H], preferred_element_type=jnp.float32)
          + jnp.dot(h1b, w2[F1H:], preferred_element_type=jnp.float32))
    y2_ref[...] = dis * xw


def _tc2(agg1, y1, dis, b1, W2):
    return pl.pallas_call(
        _tc2_body,
        grid=(GRID,),
        in_specs=[
            pl.BlockSpec((NC, BLK, F1H), lambda i: (0, i, 0)),
            pl.BlockSpec((NC, BLK, F1H), lambda i: (0, i, 0)),
            pl.BlockSpec((BLK, 1), lambda i: (i, 0)),
            pl.BlockSpec((1, F1), lambda i: (0, 0)),
            pl.BlockSpec((F1, F2), lambda i: (0, 0)),
        ],
        out_specs=pl.BlockSpec((BLK, F2), lambda i: (i, 0)),
        out_shape=jax.ShapeDtypeStruct((N, F2), jnp.float32),
    )(agg1, y1, dis, b1, W2)


def _tc3_body(agg_ref, y2_ref, dis_ref, b2_ref,
              wi1_ref, bi1_ref, wi2_ref, bi2_ref,
              wc1_ref, bc1_ref, wc2_ref, bc2_ref,
              z_ref, c_ref):
    a = agg_ref[...]
    dis = dis_ref[...]
    h = jax.nn.relu(dis * (a[0] + a[1] + y2_ref[...]) + b2_ref[...])

    t = jax.nn.relu(jnp.dot(h, wi1_ref[...], preferred_element_type=jnp.float32)
                    + bi1_ref[...])
    zi = jnp.dot(t, wi2_ref[...], preferred_element_type=jnp.float32) + bi2_ref[...]
    nrm = jnp.sqrt(jnp.sum(zi * zi, axis=1, keepdims=True))
    z_ref[...] = zi / jnp.maximum(nrm, 1e-12)

    u = jax.nn.relu(jnp.dot(h, wc1_ref[...], preferred_element_type=jnp.float32)
                    + bc1_ref[...])
    lg = jnp.dot(u, wc2_ref[...], preferred_element_type=jnp.float32) + bc2_ref[...]
    m = jnp.max(lg, axis=1, keepdims=True)
    ex = jnp.exp(lg - m)
    c_ref[...] = ex / jnp.sum(ex, axis=1, keepdims=True)


def _tc3(agg2, y2, dis, b2, Wi1, bi1, Wi2, bi2, Wc1, bc1, Wc2, bc2):
    return pl.pallas_call(
        _tc3_body,
        grid=(GRID,),
        in_specs=[
            pl.BlockSpec((NC, BLK, F2), lambda i: (0, i, 0)),
            pl.BlockSpec((BLK, F2), lambda i: (i, 0)),
            pl.BlockSpec((BLK, 1), lambda i: (i, 0)),
            pl.BlockSpec((1, F2), lambda i: (0, 0)),
            pl.BlockSpec((F2, PROHID), lambda i: (0, 0)),
            pl.BlockSpec((1, PROHID), lambda i: (0, 0)),
            pl.BlockSpec((PROHID, F2), lambda i: (0, 0)),
            pl.BlockSpec((1, F2), lambda i: (0, 0)),
            pl.BlockSpec((F2, PROHID), lambda i: (0, 0)),
            pl.BlockSpec((1, PROHID), lambda i: (0, 0)),
            pl.BlockSpec((PROHID, NCLASS), lambda i: (0, 0)),
            pl.BlockSpec((1, NCLASS), lambda i: (0, 0)),
        ],
        out_specs=[
            pl.BlockSpec((BLK, F2), lambda i: (i, 0)),
            pl.BlockSpec((BLK, NCLASS), lambda i: (i, 0)),
        ],
        out_shape=[
            jax.ShapeDtypeStruct((N, F2), jnp.float32),
            jax.ShapeDtypeStruct((N, NCLASS), jnp.float32),
        ],
    )(agg2, y2, dis, b2, Wi1, bi1, Wi2, bi2, Wc1, bc1, Wc2, bc2)


@jax.jit
def kernel(x, edge_index, W1, b1, W2, b2, Wi1, bi1, Wi2, bi2, Wc1, bc1, Wc2, bc2):
    src = edge_index[0]
    dst = edge_index[1]
    src2d = src.reshape(E // K, K)
    dst2d = dst.reshape(E // K, K)

    deg_p = _sc_deg(dst)
    deg_col = deg_p[:, :N, None]

    y1, dis = _tc1(deg_col, x, W1)
    agg1 = _sc_agg1(y1, src2d, dst2d)[:, :N]
    y2 = _tc2(agg1, y1, dis, b1.reshape(1, F1), W2)
    agg2 = _sc_agg2(y2, src2d, dst2d)[:, :N]
    z, c = _tc3(agg2, y2, dis, b2.reshape(1, F2),
                Wi1, bi1.reshape(1, PROHID), Wi2, bi2.reshape(1, F2),
                Wc1, bc1.reshape(1, PROHID), Wc2, bc2.reshape(1, NCLASS))
    return (z, c)


# speedup vs baseline: 33.4256x; 1.0882x over previous
import functools

import jax
import jax.numpy as jnp
from jax import lax
from jax.experimental import pallas as pl
from jax.experimental.pallas import tpu as pltpu
from jax.experimental.pallas import tpu_sc as plsc

N = 10000
E = 320000
NFEAT = 128
NHID = 64
F1 = 2 * NHID
F1H = F1 // 2
F2 = NHID
PROHID = 128
NCLASS = 16

NC = 2
NS = 16
NB = 8
PD = 4
NT = NC * NS
K = 80
N_PAD = 10240
RPT = N_PAD // NS

_mesh = plsc.VectorSubcoreMesh(core_axis_name="c", subcore_axis_name="s")


EPT32 = E // NT


@functools.partial(
    pl.kernel,
    out_type=jax.ShapeDtypeStruct((NC, N_PAD), jnp.float32),
    mesh=_mesh,
    scratch_types=[
        pltpu.VMEM((N_PAD,), jnp.float32),
        pltpu.VMEM((EPT32,), jnp.int32),
        pltpu.VMEM((RPT,), jnp.float32),
        pltpu.VMEM((RPT,), jnp.float32),
        pltpu.VMEM_SHARED((NS, N_PAD), jnp.float32),
    ],
    compiler_params=pltpu.CompilerParams(needs_layout_passes=False),
)
def _sc_deg(dst_hbm, out_hbm, hist, idxv, acc, tmp, spm):
    c = lax.axis_index("c")
    s = lax.axis_index("s")
    w = c * NS + s

    zero16 = jnp.zeros((16,), jnp.float32)
    one16 = jnp.full((16,), 1.0, jnp.float32)

    @pl.loop(0, N_PAD // 16)
    def _(i):
        hist[pl.ds(i * 16, 16)] = zero16

    pltpu.sync_copy(dst_hbm.at[pl.ds(w * EPT32, EPT32)], idxv)

    @pl.loop(0, EPT32 // 16)
    def _(i):
        idx = idxv[pl.ds(i * 16, 16)]
        plsc.addupdate_scatter(hist, [idx], one16)

    pltpu.sync_copy(hist, spm.at[s])
    plsc.subcore_barrier()

    @pl.loop(0, RPT // 16)
    def _(i):
        acc[pl.ds(i * 16, 16)] = zero16

    for t in range(NS):
        pltpu.sync_copy(spm.at[t, pl.ds(s * RPT, RPT)], tmp)

        @pl.loop(0, RPT // 16)
        def _(i):
            sl = pl.ds(i * 16, 16)
            acc[sl] = acc[sl] + tmp[sl]

    pltpu.sync_copy(acc, out_hbm.at[c, pl.ds(s * RPT, RPT)])


def _make_sc_agg(Fh, col_split):
    ept = E // NS if col_split else E // NT
    nch = ept // K
    NPA = N
    RPTA = NPA // NS
    ZR = 125
    CPT = N // NS

    @functools.partial(
        pl.kernel,
        out_type=jax.ShapeDtypeStruct((NC, N, Fh), jnp.float32),
        mesh=_mesh,
        scratch_types=[
            pltpu.VMEM((nch, K), jnp.int32),
            pltpu.VMEM((nch, K), jnp.int32),
            *[pltpu.VMEM((K, Fh), jnp.float32) for _ in range(NB)],
            pltpu.VMEM((ZR, Fh), jnp.float32),
            pltpu.VMEM_SHARED((NPA, Fh), jnp.float32),
            *[pltpu.SemaphoreType.DMA for _ in range(2 * NB)],
        ],
        compiler_params=pltpu.CompilerParams(use_tc_tiling_on_sc=False),
    )
    def _sc_agg(y_hbm, src2d_hbm, dst2d_hbm, out_hbm, sidx, didx, *rest):
        c = lax.axis_index("c")
        s = lax.axis_index("s")
        if col_split:
            rowbase = s * nch
            ytab = y_hbm.at[c]
        else:
            rowbase = (c * NS + s) * nch
            ytab = y_hbm

        rows = rest[:NB]
        zbuf = rest[NB]
        acc = rest[NB + 1]
        sg = rest[NB + 2:NB + 2 + NB]
        ss = rest[NB + 2 + NB:]

        pltpu.sync_copy(src2d_hbm.at[pl.ds(rowbase, nch)], sidx)
        pltpu.sync_copy(dst2d_hbm.at[pl.ds(rowbase, nch)], didx)

        zero16 = jnp.zeros((16,), jnp.float32)

        @pl.loop(0, ZR)
        def _(i):
            for j in range(Fh // 16):
                zbuf[i, pl.ds(j * 16, 16)] = zero16

        for j in range(RPTA // ZR):
            pltpu.sync_copy(zbuf, acc.at[pl.ds(s * RPTA + j * ZR, ZR)])
        plsc.subcore_barrier()

        def start_gather(g, b):
            pltpu.async_copy(ytab.at[sidx.at[g]], rows[b], sg[b])

        def wait_gather(g, b):
            pltpu.make_async_copy(ytab.at[sidx.at[g]], rows[b], sg[b]).wait()

        def start_scatter(g, b):
            pltpu.async_copy(rows[b], acc.at[didx.at[g]], ss[b], add=True)

        def wait_scatter(g, b):
            pltpu.make_async_copy(rows[b], acc.at[didx.at[g]], ss[b]).wait()

        for g in range(PD):
            start_gather(g, g)

        def body(g, b, wait_prev, prefetch):
            wait_gather(g, b)
            start_scatter(g, b)
            if prefetch:
                nb = (b + PD) % NB
                if wait_prev:
                    wait_scatter(g - (NB - PD), nb)
                start_gather(g + PD, nb)

        head = NB - PD
        for g in range(head):
            body(g, g % NB, False, True)

        ngroups = (nch - head - (NB - PD)) // NB
        tail_lo = head + NB * ngroups

        @pl.loop(0, ngroups)
        def _(gg):
            g0 = gg * NB + head
            for j in range(NB):
                body(g0 + j, (head + j) % NB, True, True)

        for g in range(tail_lo, nch):
            body(g, g % NB, True, g + PD < nch)

        for g in range(max(0, nch - NB), nch):
            wait_scatter(g, g % NB)

        plsc.subcore_barrier()
        pltpu.sync_copy(acc.at[pl.ds(s * CPT, CPT)],
                        out_hbm.at[c, pl.ds(s * CPT, CPT)])

    return _sc_agg


_sc_agg1 = _make_sc_agg(F1H, col_split=True)
_sc_agg2 = _make_sc_agg(F2, col_split=False)


BLK = 1000
GRID = N // BLK


def _tc1_body(deg_ref, x_ref, w_ref, y_ref, dis_ref):
    d = deg_ref[...]
    dis = lax.rsqrt(d[0] + d[1] + 1.0)
    xw = jnp.dot(x_ref[...], w_ref[...], preferred_element_type=jnp.float32)
    y = dis * xw
    y_ref[0] = y[:, :F1H]
    y_ref[1] = y[:, F1H:]
    dis_ref[...] = dis


def _tc1(deg_col, x, W1):
    return pl.pallas_call(
        _tc1_body,
        grid=(GRID,),
        in_specs=[
            pl.BlockSpec((NC, BLK, 1), lambda i: (0, i, 0)),
            pl.BlockSpec((BLK, NFEAT), lambda i: (i, 0)),
            pl.BlockSpec((NFEAT, F1), lambda i: (0, 0)),
        ],
        out_specs=[
            pl.BlockSpec((NC, BLK, F1H), lambda i: (0, i, 0)),
            pl.BlockSpec((BLK, 1), lambda i: (i, 0)),
        ],
        out_shape=[
            jax.ShapeDtypeStruct((NC, N, F1H), jnp.float32),
            jax.ShapeDtypeStruct((N, 1), jnp.float32),
        ],
    )(deg_col, x, W1)


def _tc2_body(agg_ref, y1_ref, dis_ref, b1_ref, w2_ref, y2_ref):
    a = agg_ref[...]
    y = y1_ref[...]
    dis = dis_ref[...]
    b1 = b1_ref[...]
    h1a = jax.nn.relu(dis * (a[0] + y[0]) + b1[:, :F1H])
    h1b = jax.nn.relu(dis * (a[1] + y[1]) + b1[:, F1H:])
    w2 = w2_ref[...]
    xw = (jnp.dot(h1a, w2[:F1H], preferred_element_type=jnp.float32)
          + jnp.dot(h1b, w2[F1H:], preferred_element_type=jnp.float32))
    y2_ref[...] = dis * xw


def _tc2(agg1, y1, dis, b1, W2):
    return pl.pallas_call(
        _tc2_body,
        grid=(GRID,),
        in_specs=[
            pl.BlockSpec((NC, BLK, F1H), lambda i: (0, i, 0)),
            pl.BlockSpec((NC, BLK, F1H), lambda i: (0, i, 0)),
            pl.BlockSpec((BLK, 1), lambda i: (i, 0)),
            pl.BlockSpec((1, F1), lambda i: (0, 0)),
            pl.BlockSpec((F1, F2), lambda i: (0, 0)),
        ],
        out_specs=pl.BlockSpec((BLK, F2), lambda i: (i, 0)),
        out_shape=jax.ShapeDtypeStruct((N, F2), jnp.float32),
    )(agg1, y1, dis, b1, W2)


def _tc3_body(agg_ref, y2_ref, dis_ref, b2_ref,
              wi1_ref, bi1_ref, wi2_ref, bi2_ref,
              wc1_ref, bc1_ref, wc2_ref, bc2_ref,
              z_ref, c_ref):
    a = agg_ref[...]
    dis = dis_ref[...]
    h = jax.nn.relu(dis * (a[0] + a[1] + y2_ref[...]) + b2_ref[...])

    t = jax.nn.relu(jnp.dot(h, wi1_ref[...], preferred_element_type=jnp.float32)
                    + bi1_ref[...])
    zi = jnp.dot(t, wi2_ref[...], preferred_element_type=jnp.float32) + bi2_ref[...]
    nrm = jnp.sqrt(jnp.sum(zi * zi, axis=1, keepdims=True))
    z_ref[...] = zi / jnp.maximum(nrm, 1e-12)

    u = jax.nn.relu(jnp.dot(h, wc1_ref[...], preferred_element_type=jnp.float32)
                    + bc1_ref[...])
    lg = jnp.dot(u, wc2_ref[...], preferred_element_type=jnp.float32) + bc2_ref[...]
    m = jnp.max(lg, axis=1, keepdims=True)
    ex = jnp.exp(lg - m)
    c_ref[...] = ex / jnp.sum(ex, axis=1, keepdims=True)


def _tc3(agg2, y2, dis, b2, Wi1, bi1, Wi2, bi2, Wc1, bc1, Wc2, bc2):
    return pl.pallas_call(
        _tc3_body,
        grid=(GRID,),
        in_specs=[
            pl.BlockSpec((NC, BLK, F2), lambda i: (0, i, 0)),
            pl.BlockSpec((BLK, F2), lambda i: (i, 0)),
            pl.BlockSpec((BLK, 1), lambda i: (i, 0)),
            pl.BlockSpec((1, F2), lambda i: (0, 0)),
            pl.BlockSpec((F2, PROHID), lambda i: (0, 0)),
            pl.BlockSpec((1, PROHID), lambda i: (0, 0)),
            pl.BlockSpec((PROHID, F2), lambda i: (0, 0)),
            pl.BlockSpec((1, F2), lambda i: (0, 0)),
            pl.BlockSpec((F2, PROHID), lambda i: (0, 0)),
            pl.BlockSpec((1, PROHID), lambda i: (0, 0)),
            pl.BlockSpec((PROHID, NCLASS), lambda i: (0, 0)),
            pl.BlockSpec((1, NCLASS), lambda i: (0, 0)),
        ],
        out_specs=[
            pl.BlockSpec((BLK, F2), lambda i: (i, 0)),
            pl.BlockSpec((BLK, NCLASS), lambda i: (i, 0)),
        ],
        out_shape=[
            jax.ShapeDtypeStruct((N, F2), jnp.float32),
            jax.ShapeDtypeStruct((N, NCLASS), jnp.float32),
        ],
    )(agg2, y2, dis, b2, Wi1, bi1, Wi2, bi2, Wc1, bc1, Wc2, bc2)


@jax.jit
def kernel(x, edge_index, W1, b1, W2, b2, Wi1, bi1, Wi2, bi2, Wc1, bc1, Wc2, bc2):
    src = edge_index[0]
    dst = edge_index[1]
    src2d = src.reshape(E // K, K)
    dst2d = dst.reshape(E // K, K)

    deg_p = _sc_deg(dst)
    deg_col = deg_p[:, :N, None]

    y1, dis = _tc1(deg_col, x, W1)
    agg1 = _sc_agg1(y1, src2d, dst2d)
    y2 = _tc2(agg1, y1, dis, b1.reshape(1, F1), W2)
    agg2 = _sc_agg2(y2, src2d, dst2d)
    z, c = _tc3(agg2, y2, dis, b2.reshape(1, F2),
                Wi1, bi1.reshape(1, PROHID), Wi2, bi2.reshape(1, F2),
                Wc1, bc1.reshape(1, PROHID), Wc2, bc2.reshape(1, NCLASS))
    return (z, c)
